# Initial kernel scaffold; baseline (speedup 1.0000x reference)
import jax
import jax.numpy as jnp
from jax.experimental import pallas as pl


def kernel(x, edge_index, W1, b1, W2, b2):
    raise NotImplementedError("write your pallas kernel here")


# speedup vs baseline: 63.3323x; 63.3323x over previous
import functools

import jax
import jax.numpy as jnp
from jax import lax
from jax.experimental import pallas as pl
from jax.experimental.pallas import tpu as pltpu
from jax.experimental.pallas import tpu_sc as plsc

N_NODES = 100000
N_EDGES = 6400000

NC = 2
NS = 16
NW = NC * NS

CHUNK = 128
KCH = 8
SB = CHUNK * KCH
NSB = N_EDGES // SB

NPAD = 100096
RPS = NPAD // NS

_mesh = plsc.VectorSubcoreMesh(core_axis_name="c", subcore_axis_name="s")


def _worker_id(c, s):
    return c * NS + s


def _deg_kernel(dst_hbm, ones_hbm, zeros_hbm, out_hbm, idx_v, ones_v, acc,
                sem):
    c = lax.axis_index("c")
    s = lax.axis_index("s")
    wid = _worker_id(c, s)

    pltpu.sync_copy(ones_hbm, ones_v)

    lo = s * RPS
    pltpu.sync_copy(zeros_hbm.at[pl.ds(lo, RPS)], acc.at[pl.ds(lo, RPS)])
    plsc.subcore_barrier()

    sb_lo = wid * NSB // NW
    sb_hi = (wid + 1) * NSB // NW

    def body(sb, _):
        pltpu.sync_copy(dst_hbm.at[sb], idx_v)
        handles = []
        for j in range(KCH):
            handles.append(
                pltpu.async_copy(ones_v, acc.at[idx_v.at[j]], sem, add=True)
            )
        for h in handles:
            h.wait()
        return ()

    lax.fori_loop(sb_lo, sb_hi, body, (), unroll=False)

    plsc.subcore_barrier()
    pltpu.sync_copy(acc.at[pl.ds(lo, RPS)], out_hbm.at[c, pl.ds(lo, RPS)])


def _sc_degree(dst3, ones1, zeros1):
    return pl.kernel(
        _deg_kernel,
        out_type=jax.ShapeDtypeStruct((NC, NPAD, 1), jnp.float32),
        mesh=_mesh,
        scratch_types=[
            pltpu.VMEM((KCH, CHUNK), jnp.int32),
            pltpu.VMEM((CHUNK, 1), jnp.float32),
            pltpu.VMEM_SHARED((NPAD, 1), jnp.float32),
            pltpu.SemaphoreType.DMA,
        ],
        compiler_params=pltpu.CompilerParams(use_tc_tiling_on_sc=False),
    )(dst3, ones1, zeros1)


def _make_edge_kernel(d):
    def body(src_hbm, dst_hbm, tab_hbm, zeros_hbm, out_hbm,
             isrc_v, idst_v, rows_v, acc, gsem, ssem):
        c = lax.axis_index("c")
        s = lax.axis_index("s")
        wid = _worker_id(c, s)

        lo = s * RPS
        pltpu.sync_copy(zeros_hbm.at[pl.ds(lo, RPS)], acc.at[pl.ds(lo, RPS)])
        plsc.subcore_barrier()

        sb_lo = wid * NSB // NW
        sb_hi = (wid + 1) * NSB // NW

        def loop(sb, _):
            pltpu.sync_copy(src_hbm.at[sb], isrc_v)
            pltpu.sync_copy(dst_hbm.at[sb], idst_v)
            gh = []
            for j in range(KCH):
                gh.append(
                    pltpu.async_copy(tab_hbm.at[isrc_v.at[j]], rows_v.at[j],
                                     gsem)
                )
            for h in gh:
                h.wait()
            sh = []
            for j in range(KCH):
                sh.append(
                    pltpu.async_copy(rows_v.at[j], acc.at[idst_v.at[j]],
                                     ssem, add=True)
                )
            for h in sh:
                h.wait()
            return ()

        lax.fori_loop(sb_lo, sb_hi, loop, (), unroll=False)

        plsc.subcore_barrier()
        pltpu.sync_copy(acc.at[pl.ds(lo, RPS)],
                        out_hbm.at[c, pl.ds(lo, RPS)])

    def call(src3, dst3, table, zeros2):
        return pl.kernel(
            body,
            out_type=jax.ShapeDtypeStruct((NC, NPAD, d), jnp.float32),
            mesh=_mesh,
            scratch_types=[
                pltpu.VMEM((KCH, CHUNK), jnp.int32),
                pltpu.VMEM((KCH, CHUNK), jnp.int32),
                pltpu.VMEM((KCH, CHUNK, d), jnp.float32),
                pltpu.VMEM_SHARED((NPAD, d), jnp.float32),
                pltpu.SemaphoreType.DMA,
                pltpu.SemaphoreType.DMA,
            ],
            compiler_params=pltpu.CompilerParams(use_tc_tiling_on_sc=False),
        )(src3, dst3, table, zeros2)

    return call


_edge_pass_16 = _make_edge_kernel(16)
_edge_pass_8 = _make_edge_kernel(8)


_RB = 2000


def _tc1_body(dega, degb, x, w1, dis_o, g1_o):
    deg = dega[...] + degb[...]
    dis = lax.rsqrt(deg)
    x_ = x[...]
    w = w1[...]
    h = x_[:, 0:1] * w[0:1, :]
    h = h + x_[:, 1:2] * w[1:2, :]
    h = h + x_[:, 2:3] * w[2:3, :]
    dis_o[...] = dis
    g1_o[...] = dis * h


def _tc1(dega, degb, x, w1):
    return pl.pallas_call(
        _tc1_body,
        grid=(N_NODES // _RB,),
        in_specs=[
            pl.BlockSpec((_RB, 1), lambda i: (i, 0)),
            pl.BlockSpec((_RB, 1), lambda i: (i, 0)),
            pl.BlockSpec((_RB, 3), lambda i: (i, 0)),
            pl.BlockSpec((3, 16), lambda i: (0, 0)),
        ],
        out_specs=[
            pl.BlockSpec((_RB, 1), lambda i: (i, 0)),
            pl.BlockSpec((_RB, 16), lambda i: (i, 0)),
        ],
        out_shape=[
            jax.ShapeDtypeStruct((N_NODES, 1), jnp.float32),
            jax.ShapeDtypeStruct((N_NODES, 16), jnp.float32),
        ],
    )(dega, degb, x, w1)


def _tc2_body(acca, accb, g1, dis, b1, w2, g2_o):
    d = dis[...]
    out1 = jnp.maximum(d * (acca[...] + accb[...] + g1[...]) + b1[...], 0.0)
    w = w2[...]
    h2 = out1[:, 0:1] * w[0:1, :]
    for k in range(1, 16):
        h2 = h2 + out1[:, k:k + 1] * w[k:k + 1, :]
    g2_o[...] = d * h2


def _tc2(acca, accb, g1, dis, b1, w2p):
    return pl.pallas_call(
        _tc2_body,
        grid=(N_NODES // _RB,),
        in_specs=[
            pl.BlockSpec((_RB, 16), lambda i: (i, 0)),
            pl.BlockSpec((_RB, 16), lambda i: (i, 0)),
            pl.BlockSpec((_RB, 16), lambda i: (i, 0)),
            pl.BlockSpec((_RB, 1), lambda i: (i, 0)),
            pl.BlockSpec((1, 16), lambda i: (0, 0)),
            pl.BlockSpec((16, 8), lambda i: (0, 0)),
        ],
        out_specs=pl.BlockSpec((_RB, 8), lambda i: (i, 0)),
        out_shape=jax.ShapeDtypeStruct((N_NODES, 8), jnp.float32),
    )(acca, accb, g1, dis, b1, w2p)


def _tc3_body(acca, accb, g2, dis, b2, out_o):
    out_o[...] = dis[...] * (acca[...] + accb[...] + g2[...]) + b2[...]


def _tc3(acca, accb, g2, dis, b2p):
    return pl.pallas_call(
        _tc3_body,
        grid=(N_NODES // _RB,),
        in_specs=[
            pl.BlockSpec((_RB, 8), lambda i: (i, 0)),
            pl.BlockSpec((_RB, 8), lambda i: (i, 0)),
            pl.BlockSpec((_RB, 8), lambda i: (i, 0)),
            pl.BlockSpec((_RB, 1), lambda i: (i, 0)),
            pl.BlockSpec((1, 8), lambda i: (0, 0)),
        ],
        out_specs=pl.BlockSpec((_RB, 8), lambda i: (i, 0)),
        out_shape=jax.ShapeDtypeStruct((N_NODES, 8), jnp.float32),
    )(acca, accb, g2, dis, b2p)


def kernel(x, edge_index, W1, b1, W2, b2):
    src3 = edge_index[0].reshape(NSB, KCH, CHUNK)
    dst3 = edge_index[1].reshape(NSB, KCH, CHUNK)

    zeros1 = jnp.zeros((NPAD, 1), jnp.float32)
    zeros16 = jnp.zeros((NPAD, 16), jnp.float32)
    zeros8 = jnp.zeros((NPAD, 8), jnp.float32)
    ones1 = jnp.ones((CHUNK, 1), jnp.float32)

    deg2 = _sc_degree(dst3, ones1, zeros1)
    dega = deg2[0, :N_NODES, :]
    degb = deg2[1, :N_NODES, :]

    dis, g1 = _tc1(dega, degb, x, W1)

    acc1 = _edge_pass_16(src3, dst3, g1, zeros16)
    b1r = b1.reshape(1, 16)
    w2p = jnp.pad(W2, ((0, 0), (0, 1)))
    g2 = _tc2(acc1[0, :N_NODES], acc1[1, :N_NODES], g1, dis, b1r, w2p)

    acc2 = _edge_pass_8(src3, dst3, g2, zeros8)
    b2p = jnp.pad(b2, (0, 1)).reshape(1, 8)
    out = _tc3(acc2[0, :N_NODES], acc2[1, :N_NODES], g2, dis, b2p)
    return out[:, :7]


# speedup vs baseline: 91.0357x; 1.4374x over previous
import jax
import jax.numpy as jnp
from jax import lax
from jax.experimental import pallas as pl
from jax.experimental.pallas import tpu as pltpu
from jax.experimental.pallas import tpu_sc as plsc

N_NODES = 100000
N_EDGES = 6400000

NC = 2
NS = 16
NW = NC * NS

CHUNK = 128
KCH = 5
SB = CHUNK * KCH
NSB = N_EDGES // SB

NPAD = 100096
RPS = NPAD // NS

_mesh = plsc.VectorSubcoreMesh(core_axis_name="c", subcore_axis_name="s")
_sc_params = pltpu.CompilerParams(use_tc_tiling_on_sc=False)


def _deg_kernel(dst_hbm, ones_hbm, zeros_hbm, out_hbm, idx_v, ones_v, acc,
                isem, ssem):
    c = lax.axis_index("c")
    s = lax.axis_index("s")
    wid = c * NS + s

    pltpu.sync_copy(ones_hbm, ones_v.at[0])

    lo = s * RPS
    pltpu.sync_copy(zeros_hbm.at[pl.ds(lo, RPS)], acc.at[pl.ds(lo, RPS)])
    plsc.subcore_barrier()

    sb_lo = wid * NSB // NW
    sb_hi = (wid + 1) * NSB // NW
    count = sb_hi - sb_lo

    def stage_scatter(sb, p):
        pltpu.async_copy(dst_hbm.at[sb], idx_v.at[p], isem)
        pltpu.make_async_copy(dst_hbm.at[0], idx_v.at[p], isem).wait()
        for j in range(KCH):
            pltpu.async_copy(ones_v.at[0], acc.at[idx_v.at[p, j]],
                             ssem.at[p], add=True)

    def drain_scatter(p):
        for j in range(KCH):
            pltpu.make_async_copy(ones_v.at[0], acc.at[idx_v.at[p, j]],
                                  ssem.at[p]).wait()

    def body(t, _):
        @pl.when(t >= 2)
        def _():
            drain_scatter((t - 2) % 2)

        @pl.when(t < count)
        def _():
            stage_scatter(sb_lo + t, t % 2)

        return ()

    lax.fori_loop(0, count + 2, body, (), unroll=False)

    plsc.subcore_barrier()
    pltpu.sync_copy(acc.at[pl.ds(lo, RPS)], out_hbm.at[c, pl.ds(lo, RPS)])


def _sc_degree(dst3, ones1, zeros1):
    return pl.kernel(
        _deg_kernel,
        out_type=jax.ShapeDtypeStruct((NC, NPAD, 1), jnp.float32),
        mesh=_mesh,
        scratch_types=[
            pltpu.VMEM((2, KCH, CHUNK), jnp.int32),
            pltpu.VMEM((1, CHUNK, 1), jnp.float32),
            pltpu.VMEM_SHARED((NPAD, 1), jnp.float32),
            pltpu.SemaphoreType.DMA,
            pltpu.SemaphoreType.DMA((2,)),
        ],
        compiler_params=_sc_params,
    )(dst3, ones1, zeros1)


def _make_edge_kernel(d):
    def body(src_hbm, dst_hbm, tab_hbm, zeros_hbm, out_hbm,
             isrc_v, idst_v, rows_v, acc, isem, gsem, ssem):
        c = lax.axis_index("c")
        s = lax.axis_index("s")
        wid = c * NS + s

        lo = s * RPS
        pltpu.sync_copy(zeros_hbm.at[pl.ds(lo, RPS)], acc.at[pl.ds(lo, RPS)])
        plsc.subcore_barrier()

        sb_lo = wid * NSB // NW
        sb_hi = (wid + 1) * NSB // NW
        count = sb_hi - sb_lo

        def stage_a(sb, p):
            pltpu.async_copy(src_hbm.at[sb], isrc_v.at[p], isem)
            pltpu.async_copy(dst_hbm.at[sb], idst_v.at[p], isem)
            pltpu.make_async_copy(src_hbm.at[0], isrc_v.at[p], isem).wait()
            pltpu.make_async_copy(dst_hbm.at[0], idst_v.at[p], isem).wait()
            for j in range(KCH):
                pltpu.async_copy(tab_hbm.at[isrc_v.at[p, j]],
                                 rows_v.at[p, j], gsem.at[p])

        def stage_b(p):
            for j in range(KCH):
                pltpu.make_async_copy(tab_hbm.at[isrc_v.at[p, j]],
                                      rows_v.at[p, j], gsem.at[p]).wait()
            for j in range(KCH):
                pltpu.async_copy(rows_v.at[p, j], acc.at[idst_v.at[p, j]],
                                 ssem.at[p], add=True)

        def stage_c(p):
            for j in range(KCH):
                pltpu.make_async_copy(rows_v.at[p, j],
                                      acc.at[idst_v.at[p, j]],
                                      ssem.at[p]).wait()

        def body_t(t, _):
            @pl.when(t >= 2)
            def _():
                stage_c((t - 2) % 2)

            @pl.when(t < count)
            def _():
                stage_a(sb_lo + t, t % 2)

            @pl.when((t >= 1) & (t <= count))
            def _():
                stage_b((t - 1) % 2)

            return ()

        lax.fori_loop(0, count + 2, body_t, (), unroll=False)

        plsc.subcore_barrier()
        pltpu.sync_copy(acc.at[pl.ds(lo, RPS)],
                        out_hbm.at[c, pl.ds(lo, RPS)])

    def call(src3, dst3, table, zeros2):
        return pl.kernel(
            body,
            out_type=jax.ShapeDtypeStruct((NC, NPAD, d), jnp.float32),
            mesh=_mesh,
            scratch_types=[
                pltpu.VMEM((2, KCH, CHUNK), jnp.int32),
                pltpu.VMEM((2, KCH, CHUNK), jnp.int32),
                pltpu.VMEM((2, KCH, CHUNK, d), jnp.float32),
                pltpu.VMEM_SHARED((NPAD, d), jnp.float32),
                pltpu.SemaphoreType.DMA,
                pltpu.SemaphoreType.DMA((2,)),
                pltpu.SemaphoreType.DMA((2,)),
            ],
            compiler_params=_sc_params,
        )(src3, dst3, table, zeros2)

    return call


_edge_pass_16 = _make_edge_kernel(16)
_edge_pass_8 = _make_edge_kernel(8)


_RB = 5000


def _tc1_body(deg2, x, w1, dis_o, g1_o):
    deg = deg2[0] + deg2[1]
    dis = lax.rsqrt(deg)
    x_ = x[...]
    w = w1[...]
    h = x_[:, 0:1] * w[0:1, :]
    h = h + x_[:, 1:2] * w[1:2, :]
    h = h + x_[:, 2:3] * w[2:3, :]
    dis_o[...] = dis
    g1_o[...] = dis * h


def _tc1(deg2, x, w1):
    return pl.pallas_call(
        _tc1_body,
        grid=(N_NODES // _RB,),
        in_specs=[
            pl.BlockSpec((2, _RB, 1), lambda i: (0, i, 0)),
            pl.BlockSpec((_RB, 3), lambda i: (i, 0)),
            pl.BlockSpec((3, 16), lambda i: (0, 0)),
        ],
        out_specs=[
            pl.BlockSpec((_RB, 1), lambda i: (i, 0)),
            pl.BlockSpec((_RB, 16), lambda i: (i, 0)),
        ],
        out_shape=[
            jax.ShapeDtypeStruct((N_NODES, 1), jnp.float32),
            jax.ShapeDtypeStruct((N_NODES, 16), jnp.float32),
        ],
    )(deg2, x, w1)


def _tc2_body(acc2, g1, dis, b1, w2, g2_o):
    d = dis[...]
    out1 = jnp.maximum(d * (acc2[0] + acc2[1] + g1[...]) + b1[...], 0.0)
    w = w2[...]
    h2 = out1[:, 0:1] * w[0:1, :]
    for k in range(1, 16):
        h2 = h2 + out1[:, k:k + 1] * w[k:k + 1, :]
    g2_o[...] = d * h2


def _tc2(acc2, g1, dis, b1, w2p):
    return pl.pallas_call(
        _tc2_body,
        grid=(N_NODES // _RB,),
        in_specs=[
            pl.BlockSpec((2, _RB, 16), lambda i: (0, i, 0)),
            pl.BlockSpec((_RB, 16), lambda i: (i, 0)),
            pl.BlockSpec((_RB, 1), lambda i: (i, 0)),
            pl.BlockSpec((1, 16), lambda i: (0, 0)),
            pl.BlockSpec((16, 8), lambda i: (0, 0)),
        ],
        out_specs=pl.BlockSpec((_RB, 8), lambda i: (i, 0)),
        out_shape=jax.ShapeDtypeStruct((N_NODES, 8), jnp.float32),
    )(acc2, g1, dis, b1, w2p)


def _tc3_body(acc2, g2, dis, b2, out_o):
    out_o[...] = dis[...] * (acc2[0] + acc2[1] + g2[...]) + b2[...]


def _tc3(acc2, g2, dis, b2p):
    return pl.pallas_call(
        _tc3_body,
        grid=(N_NODES // _RB,),
        in_specs=[
            pl.BlockSpec((2, _RB, 8), lambda i: (0, i, 0)),
            pl.BlockSpec((_RB, 8), lambda i: (i, 0)),
            pl.BlockSpec((_RB, 1), lambda i: (i, 0)),
            pl.BlockSpec((1, 8), lambda i: (0, 0)),
        ],
        out_specs=pl.BlockSpec((_RB, 8), lambda i: (i, 0)),
        out_shape=jax.ShapeDtypeStruct((N_NODES, 8), jnp.float32),
    )(acc2, g2, dis, b2p)


def kernel(x, edge_index, W1, b1, W2, b2):
    src3 = edge_index[0].reshape(NSB, KCH, CHUNK)
    dst3 = edge_index[1].reshape(NSB, KCH, CHUNK)

    zeros1 = jnp.zeros((NPAD, 1), jnp.float32)
    zeros16 = jnp.zeros((NPAD, 16), jnp.float32)
    zeros8 = jnp.zeros((NPAD, 8), jnp.float32)
    ones1 = jnp.ones((CHUNK, 1), jnp.float32)

    deg2 = _sc_degree(dst3, ones1, zeros1)

    dis, g1 = _tc1(deg2, x, W1)

    acc1 = _edge_pass_16(src3, dst3, g1, zeros16)
    b1r = b1.reshape(1, 16)
    w2p = jnp.pad(W2, ((0, 0), (0, 1)))
    g2 = _tc2(acc1, g1, dis, b1r, w2p)

    acc2 = _edge_pass_8(src3, dst3, g2, zeros8)
    b2p = jnp.pad(b2, (0, 1)).reshape(1, 8)
    out = _tc3(acc2, g2, dis, b2p)
    return out[:, :7]


# speedup vs baseline: 105.7684x; 1.1618x over previous
import jax
import jax.numpy as jnp
from jax import lax
from jax.experimental import pallas as pl
from jax.experimental.pallas import tpu as pltpu
from jax.experimental.pallas import tpu_sc as plsc

N_NODES = 100000
N_EDGES = 6400000

NC = 2
NS = 16
NW = NC * NS

CHUNK = 128
KCH = 5
SB = CHUNK * KCH
NSB = N_EDGES // SB

DKCH = 10
DNSB = N_EDGES // (CHUNK * DKCH)

NPAD = 100096
RPS = NPAD // NS

_mesh = plsc.VectorSubcoreMesh(core_axis_name="c", subcore_axis_name="s")
_sc_params = pltpu.CompilerParams(use_tc_tiling_on_sc=False)


def _deg_kernel(dst_hbm, ones_hbm, zeros_hbm, out_hbm, idx_v, ones_v, acc,
                isem, ssem):
    c = lax.axis_index("c")
    s = lax.axis_index("s")
    wid = c * NS + s

    pltpu.sync_copy(ones_hbm, ones_v.at[0])

    lo = s * RPS
    pltpu.sync_copy(zeros_hbm.at[pl.ds(lo, RPS)], acc.at[pl.ds(lo, RPS)])
    plsc.subcore_barrier()

    sb_lo = wid * DNSB // NW
    sb_hi = (wid + 1) * DNSB // NW
    count = sb_hi - sb_lo

    def stage_scatter(sb, p):
        pltpu.async_copy(dst_hbm.at[sb], idx_v.at[p], isem)
        pltpu.make_async_copy(dst_hbm.at[0], idx_v.at[p], isem).wait()
        for j in range(DKCH):
            pltpu.async_copy(ones_v.at[0], acc.at[idx_v.at[p, j]],
                             ssem.at[p], add=True)

    def drain_scatter(p):
        for j in range(DKCH):
            pltpu.make_async_copy(ones_v.at[0], acc.at[idx_v.at[p, j]],
                                  ssem.at[p]).wait()

    def body(t, _):
        @pl.when(t >= 2)
        def _():
            drain_scatter((t - 2) % 2)

        @pl.when(t < count)
        def _():
            stage_scatter(sb_lo + t, t % 2)

        return ()

    lax.fori_loop(0, count + 2, body, (), unroll=False)

    plsc.subcore_barrier()
    pltpu.sync_copy(acc.at[pl.ds(lo, RPS)], out_hbm.at[c, pl.ds(lo, RPS)])


def _sc_degree(dst3, ones1, zeros1):
    return pl.kernel(
        _deg_kernel,
        out_type=jax.ShapeDtypeStruct((NC, NPAD, 1), jnp.float32),
        mesh=_mesh,
        scratch_types=[
            pltpu.VMEM((2, DKCH, CHUNK), jnp.int32),
            pltpu.VMEM((1, CHUNK, 1), jnp.float32),
            pltpu.VMEM_SHARED((NPAD, 1), jnp.float32),
            pltpu.SemaphoreType.DMA,
            pltpu.SemaphoreType.DMA((2,)),
        ],
        compiler_params=_sc_params,
    )(dst3, ones1, zeros1)


def _make_edge_kernel(d):
    def body(src_hbm, dst_hbm, tab_hbm, zeros_hbm, out_hbm,
             isrc_v, idst_v, rows_v, acc, isem, gsem, ssem):
        c = lax.axis_index("c")
        s = lax.axis_index("s")
        wid = c * NS + s

        lo = s * RPS
        pltpu.sync_copy(zeros_hbm.at[pl.ds(lo, RPS)], acc.at[pl.ds(lo, RPS)])
        plsc.subcore_barrier()

        sb_lo = wid * NSB // NW
        sb_hi = (wid + 1) * NSB // NW
        count = sb_hi - sb_lo

        def stage_a(sb, p):
            pltpu.async_copy(src_hbm.at[sb], isrc_v.at[p], isem)
            pltpu.async_copy(dst_hbm.at[sb], idst_v.at[p], isem)
            pltpu.make_async_copy(src_hbm.at[0], isrc_v.at[p], isem).wait()
            pltpu.make_async_copy(dst_hbm.at[0], idst_v.at[p], isem).wait()
            for j in range(KCH):
                pltpu.async_copy(tab_hbm.at[isrc_v.at[p, j]],
                                 rows_v.at[p, j], gsem.at[p])

        def stage_b(p):
            for j in range(KCH):
                pltpu.make_async_copy(tab_hbm.at[isrc_v.at[p, j]],
                                      rows_v.at[p, j], gsem.at[p]).wait()
            for j in range(KCH):
                pltpu.async_copy(rows_v.at[p, j], acc.at[idst_v.at[p, j]],
                                 ssem.at[p], add=True)

        def stage_c(p):
            for j in range(KCH):
                pltpu.make_async_copy(rows_v.at[p, j],
                                      acc.at[idst_v.at[p, j]],
                                      ssem.at[p]).wait()

        def body_t(t, _):
            @pl.when(t >= 2)
            def _():
                stage_c((t - 2) % 2)

            @pl.when(t < count)
            def _():
                stage_a(sb_lo + t, t % 2)

            @pl.when((t >= 1) & (t <= count))
            def _():
                stage_b((t - 1) % 2)

            return ()

        lax.fori_loop(0, count + 2, body_t, (), unroll=False)

        plsc.subcore_barrier()
        pltpu.sync_copy(acc.at[pl.ds(lo, RPS)],
                        out_hbm.at[c, pl.ds(lo, RPS)])

    def call(src3, dst3, table, zeros2):
        return pl.kernel(
            body,
            out_type=jax.ShapeDtypeStruct((NC, NPAD, d), jnp.float32),
            mesh=_mesh,
            scratch_types=[
                pltpu.VMEM((2, KCH, CHUNK), jnp.int32),
                pltpu.VMEM((2, KCH, CHUNK), jnp.int32),
                pltpu.VMEM((2, KCH, CHUNK, d), jnp.float32),
                pltpu.VMEM_SHARED((NPAD, d), jnp.float32),
                pltpu.SemaphoreType.DMA,
                pltpu.SemaphoreType.DMA((2,)),
                pltpu.SemaphoreType.DMA((2,)),
            ],
            compiler_params=_sc_params,
        )(src3, dst3, table, zeros2)

    return call


_edge_pass_16 = _make_edge_kernel(16)
_edge_pass_8 = _make_edge_kernel(8)


_RB = 5000


def _tc1_body(deg2, x, w1, dis_o, g1_o):
    deg = deg2[0] + deg2[1]
    dis = lax.rsqrt(deg)
    x_ = x[...]
    w = w1[...]
    h = x_[:, 0:1] * w[0:1, :]
    h = h + x_[:, 1:2] * w[1:2, :]
    h = h + x_[:, 2:3] * w[2:3, :]
    dis_o[...] = dis
    g1_o[...] = dis * h


def _tc1(deg2, x, w1):
    return pl.pallas_call(
        _tc1_body,
        grid=(N_NODES // _RB,),
        in_specs=[
            pl.BlockSpec((2, _RB, 1), lambda i: (0, i, 0)),
            pl.BlockSpec((_RB, 3), lambda i: (i, 0)),
            pl.BlockSpec((3, 16), lambda i: (0, 0)),
        ],
        out_specs=[
            pl.BlockSpec((_RB, 1), lambda i: (i, 0)),
            pl.BlockSpec((_RB, 16), lambda i: (i, 0)),
        ],
        out_shape=[
            jax.ShapeDtypeStruct((N_NODES, 1), jnp.float32),
            jax.ShapeDtypeStruct((N_NODES, 16), jnp.float32),
        ],
    )(deg2, x, w1)


def _tc2_body(acc2, g1, dis, b1, w2, g2_o):
    d = dis[...]
    out1 = jnp.maximum(d * (acc2[0] + acc2[1] + g1[...]) + b1[...], 0.0)
    h2 = jnp.dot(out1, w2[...], preferred_element_type=jnp.float32)
    g2_o[...] = d * h2


def _tc2(acc2, g1, dis, b1, w2p):
    return pl.pallas_call(
        _tc2_body,
        grid=(N_NODES // _RB,),
        in_specs=[
            pl.BlockSpec((2, _RB, 16), lambda i: (0, i, 0)),
            pl.BlockSpec((_RB, 16), lambda i: (i, 0)),
            pl.BlockSpec((_RB, 1), lambda i: (i, 0)),
            pl.BlockSpec((1, 16), lambda i: (0, 0)),
            pl.BlockSpec((16, 8), lambda i: (0, 0)),
        ],
        out_specs=pl.BlockSpec((_RB, 8), lambda i: (i, 0)),
        out_shape=jax.ShapeDtypeStruct((N_NODES, 8), jnp.float32),
    )(acc2, g1, dis, b1, w2p)


def _tc3_body(acc2, g2, dis, b2, out_o):
    out_o[...] = dis[...] * (acc2[0] + acc2[1] + g2[...]) + b2[...]


def _tc3(acc2, g2, dis, b2p):
    return pl.pallas_call(
        _tc3_body,
        grid=(N_NODES // _RB,),
        in_specs=[
            pl.BlockSpec((2, _RB, 8), lambda i: (0, i, 0)),
            pl.BlockSpec((_RB, 8), lambda i: (i, 0)),
            pl.BlockSpec((_RB, 1), lambda i: (i, 0)),
            pl.BlockSpec((1, 8), lambda i: (0, 0)),
        ],
        out_specs=pl.BlockSpec((_RB, 8), lambda i: (i, 0)),
        out_shape=jax.ShapeDtypeStruct((N_NODES, 8), jnp.float32),
    )(acc2, g2, dis, b2p)


def kernel(x, edge_index, W1, b1, W2, b2):
    src3 = edge_index[0].reshape(NSB, KCH, CHUNK)
    dst3 = edge_index[1].reshape(NSB, KCH, CHUNK)

    zeros1 = jnp.zeros((NPAD, 1), jnp.float32)
    zeros16 = jnp.zeros((NPAD, 16), jnp.float32)
    zeros8 = jnp.zeros((NPAD, 8), jnp.float32)
    ones1 = jnp.ones((CHUNK, 1), jnp.float32)

    dstd = edge_index[1].reshape(DNSB, DKCH, CHUNK)
    deg2 = _sc_degree(dstd, ones1, zeros1)

    dis, g1 = _tc1(deg2, x, W1)

    acc1 = _edge_pass_16(src3, dst3, g1, zeros16)
    b1r = b1.reshape(1, 16)
    w2p = jnp.pad(W2, ((0, 0), (0, 1)))
    g2 = _tc2(acc1, g1, dis, b1r, w2p)

    acc2 = _edge_pass_8(src3, dst3, g2, zeros8)
    b2p = jnp.pad(b2, (0, 1)).reshape(1, 8)
    out = _tc3(acc2, g2, dis, b2p)
    return out[:, :7]


# speedup vs baseline: 106.8971x; 1.0107x over previous
import jax
import jax.numpy as jnp
from jax import lax
from jax.experimental import pallas as pl
from jax.experimental.pallas import tpu as pltpu
from jax.experimental.pallas import tpu_sc as plsc

N_NODES = 100000
N_EDGES = 6400000

NC = 2
NS = 16
NW = NC * NS

CHUNK = 128
KCH = 5
SB = CHUNK * KCH
NSB = N_EDGES // SB

NCHUNKS = N_EDGES // CHUNK

DKCH = 10
DNSB = N_EDGES // (CHUNK * DKCH)

NPAD = 100096
RPS = NPAD // NS

_mesh = plsc.VectorSubcoreMesh(core_axis_name="c", subcore_axis_name="s")
_sc_params = pltpu.CompilerParams(use_tc_tiling_on_sc=False)


def _deg_kernel(dst_hbm, ones_hbm, zeros_hbm, out_hbm, idx_v, ones_v, acc,
                isem, ssem):
    c = lax.axis_index("c")
    s = lax.axis_index("s")
    wid = c * NS + s

    pltpu.sync_copy(ones_hbm, ones_v.at[0])

    lo = s * RPS
    for q in range(8):
        pltpu.sync_copy(zeros_hbm, acc.at[pl.ds(lo + q * (RPS // 8),
                                                RPS // 8)])
    plsc.subcore_barrier()

    sb_lo = wid * DNSB // NW
    sb_hi = (wid + 1) * DNSB // NW
    count = sb_hi - sb_lo

    def stage_scatter(sb, p):
        pltpu.async_copy(dst_hbm.at[pl.ds(sb * DKCH, DKCH)], idx_v.at[p],
                         isem)
        pltpu.make_async_copy(dst_hbm.at[pl.ds(0, DKCH)], idx_v.at[p],
                              isem).wait()
        for j in range(DKCH):
            pltpu.async_copy(ones_v.at[0], acc.at[idx_v.at[p, j]],
                             ssem.at[p], add=True)

    def drain_scatter(p):
        for j in range(DKCH):
            pltpu.make_async_copy(ones_v.at[0], acc.at[idx_v.at[p, j]],
                                  ssem.at[p]).wait()

    def body(t, _):
        @pl.when(t >= 2)
        def _():
            drain_scatter((t - 2) % 2)

        @pl.when(t < count)
        def _():
            stage_scatter(sb_lo + t, t % 2)

        return ()

    lax.fori_loop(0, count + 2, body, (), unroll=False)

    plsc.subcore_barrier()
    pltpu.sync_copy(acc.at[pl.ds(lo, RPS)], out_hbm.at[c, pl.ds(lo, RPS)])


def _sc_degree(dst3, ones1, zeros1):
    return pl.kernel(
        _deg_kernel,
        out_type=jax.ShapeDtypeStruct((NC, NPAD, 1), jnp.float32),
        mesh=_mesh,
        scratch_types=[
            pltpu.VMEM((2, DKCH, CHUNK), jnp.int32),
            pltpu.VMEM((1, CHUNK, 1), jnp.float32),
            pltpu.VMEM_SHARED((NPAD, 1), jnp.float32),
            pltpu.SemaphoreType.DMA,
            pltpu.SemaphoreType.DMA((2,)),
        ],
        compiler_params=_sc_params,
    )(dst3, ones1, zeros1)


def _make_edge_kernel(d):
    def body(src_hbm, dst_hbm, tab_hbm, zeros_hbm, out_hbm,
             isrc_v, idst_v, rows_v, acc, isem, gsem, ssem):
        c = lax.axis_index("c")
        s = lax.axis_index("s")
        wid = c * NS + s

        lo = s * RPS
        for q in range(8):
            pltpu.sync_copy(zeros_hbm, acc.at[pl.ds(lo + q * (RPS // 8),
                                                    RPS // 8)])
        plsc.subcore_barrier()

        sb_lo = wid * NSB // NW
        sb_hi = (wid + 1) * NSB // NW
        count = sb_hi - sb_lo

        def stage_a(sb, p):
            pltpu.async_copy(src_hbm.at[pl.ds(sb * KCH, KCH)],
                             isrc_v.at[p], isem)
            pltpu.async_copy(dst_hbm.at[pl.ds(sb * KCH, KCH)],
                             idst_v.at[p], isem)
            pltpu.make_async_copy(src_hbm.at[pl.ds(0, KCH)], isrc_v.at[p],
                                  isem).wait()
            pltpu.make_async_copy(dst_hbm.at[pl.ds(0, KCH)], idst_v.at[p],
                                  isem).wait()
            for j in range(KCH):
                pltpu.async_copy(tab_hbm.at[isrc_v.at[p, j]],
                                 rows_v.at[p, j], gsem.at[p])

        def stage_b(p):
            for j in range(KCH):
                pltpu.make_async_copy(tab_hbm.at[isrc_v.at[p, j]],
                                      rows_v.at[p, j], gsem.at[p]).wait()
            for j in range(KCH):
                pltpu.async_copy(rows_v.at[p, j], acc.at[idst_v.at[p, j]],
                                 ssem.at[p], add=True)

        def stage_c(p):
            for j in range(KCH):
                pltpu.make_async_copy(rows_v.at[p, j],
                                      acc.at[idst_v.at[p, j]],
                                      ssem.at[p]).wait()

        def body_t(t, _):
            @pl.when(t >= 2)
            def _():
                stage_c((t - 2) % 2)

            @pl.when(t < count)
            def _():
                stage_a(sb_lo + t, t % 2)

            @pl.when((t >= 1) & (t <= count))
            def _():
                stage_b((t - 1) % 2)

            return ()

        lax.fori_loop(0, count + 2, body_t, (), unroll=False)

        plsc.subcore_barrier()
        pltpu.sync_copy(acc.at[pl.ds(lo, RPS)],
                        out_hbm.at[c, pl.ds(lo, RPS)])

    def call(src3, dst3, table, zeros2):
        return pl.kernel(
            body,
            out_type=jax.ShapeDtypeStruct((NC, NPAD, d), jnp.float32),
            mesh=_mesh,
            scratch_types=[
                pltpu.VMEM((2, KCH, CHUNK), jnp.int32),
                pltpu.VMEM((2, KCH, CHUNK), jnp.int32),
                pltpu.VMEM((2, KCH, CHUNK, d), jnp.float32),
                pltpu.VMEM_SHARED((NPAD, d), jnp.float32),
                pltpu.SemaphoreType.DMA,
                pltpu.SemaphoreType.DMA((2,)),
                pltpu.SemaphoreType.DMA((2,)),
            ],
            compiler_params=_sc_params,
        )(src3, dst3, table, zeros2)

    return call


_edge_pass_16 = _make_edge_kernel(16)
_edge_pass_8 = _make_edge_kernel(8)


_RB = 5000


def _tc1_body(deg2, x, w1, dis_o, g1_o):
    deg = deg2[0] + deg2[1]
    dis = lax.rsqrt(deg)
    x_ = x[...]
    w = w1[...]
    h = x_[:, 0:1] * w[0:1, :]
    h = h + x_[:, 1:2] * w[1:2, :]
    h = h + x_[:, 2:3] * w[2:3, :]
    dis_o[...] = dis
    g1_o[...] = dis * h


def _tc1(deg2, x, w1):
    return pl.pallas_call(
        _tc1_body,
        grid=(N_NODES // _RB,),
        in_specs=[
            pl.BlockSpec((2, _RB, 1), lambda i: (0, i, 0)),
            pl.BlockSpec((_RB, 3), lambda i: (i, 0)),
            pl.BlockSpec((3, 16), lambda i: (0, 0)),
        ],
        out_specs=[
            pl.BlockSpec((_RB, 1), lambda i: (i, 0)),
            pl.BlockSpec((_RB, 16), lambda i: (i, 0)),
        ],
        out_shape=[
            jax.ShapeDtypeStruct((N_NODES, 1), jnp.float32),
            jax.ShapeDtypeStruct((N_NODES, 16), jnp.float32),
        ],
    )(deg2, x, w1)


def _tc2_body(acc2, g1, dis, b1, w2, g2_o):
    d = dis[...]
    out1 = jnp.maximum(d * (acc2[0] + acc2[1] + g1[...]) + b1[...], 0.0)
    h2 = jnp.dot(out1, w2[...], preferred_element_type=jnp.float32)
    g2_o[...] = d * h2


def _tc2(acc2, g1, dis, b1, w2p):
    return pl.pallas_call(
        _tc2_body,
        grid=(N_NODES // _RB,),
        in_specs=[
            pl.BlockSpec((2, _RB, 16), lambda i: (0, i, 0)),
            pl.BlockSpec((_RB, 16), lambda i: (i, 0)),
            pl.BlockSpec((_RB, 1), lambda i: (i, 0)),
            pl.BlockSpec((1, 16), lambda i: (0, 0)),
            pl.BlockSpec((16, 8), lambda i: (0, 0)),
        ],
        out_specs=pl.BlockSpec((_RB, 8), lambda i: (i, 0)),
        out_shape=jax.ShapeDtypeStruct((N_NODES, 8), jnp.float32),
    )(acc2, g1, dis, b1, w2p)


def _tc3_body(acc2, g2, dis, b2, out_o):
    out_o[...] = dis[...] * (acc2[0] + acc2[1] + g2[...]) + b2[...]


def _tc3(acc2, g2, dis, b2p):
    return pl.pallas_call(
        _tc3_body,
        grid=(N_NODES // _RB,),
        in_specs=[
            pl.BlockSpec((2, _RB, 8), lambda i: (0, i, 0)),
            pl.BlockSpec((_RB, 8), lambda i: (i, 0)),
            pl.BlockSpec((_RB, 1), lambda i: (i, 0)),
            pl.BlockSpec((1, 8), lambda i: (0, 0)),
        ],
        out_specs=pl.BlockSpec((_RB, 8), lambda i: (i, 0)),
        out_shape=jax.ShapeDtypeStruct((N_NODES, 8), jnp.float32),
    )(acc2, g2, dis, b2p)


def kernel(x, edge_index, W1, b1, W2, b2):
    src3 = edge_index[0].reshape(NCHUNKS, CHUNK)
    dst3 = edge_index[1].reshape(NCHUNKS, CHUNK)

    zeros1 = jnp.zeros((RPS // 8, 1), jnp.float32)
    zeros16 = jnp.zeros((RPS // 8, 16), jnp.float32)
    zeros8 = jnp.zeros((RPS // 8, 8), jnp.float32)
    ones1 = jnp.ones((CHUNK, 1), jnp.float32)

    deg2 = _sc_degree(dst3, ones1, zeros1)

    dis, g1 = _tc1(deg2, x, W1)

    acc1 = _edge_pass_16(src3, dst3, g1, zeros16)
    b1r = b1.reshape(1, 16)
    w2p = jnp.pad(W2, ((0, 0), (0, 1)))
    g2 = _tc2(acc1, g1, dis, b1r, w2p)

    acc2 = _edge_pass_8(src3, dst3, g2, zeros8)
    b2p = jnp.pad(b2, (0, 1)).reshape(1, 8)
    out = _tc3(acc2, g2, dis, b2p)
    return out[:, :7]


# speedup vs baseline: 116.5679x; 1.0905x over previous
import jax
import jax.numpy as jnp
from jax import lax
from jax.experimental import pallas as pl
from jax.experimental.pallas import tpu as pltpu
from jax.experimental.pallas import tpu_sc as plsc

N_NODES = 100000
N_EDGES = 6400000

NC = 2
NS = 16
NW = NC * NS

CHUNK = 128
KCH = 5
SB = CHUNK * KCH
NSB = N_EDGES // SB

NCHUNKS = N_EDGES // CHUNK

DKCH = 20
DNSB = N_EDGES // (CHUNK * DKCH)

NPAD = 100096
RPS = NPAD // NS

_mesh = plsc.VectorSubcoreMesh(core_axis_name="c", subcore_axis_name="s")
_sc_params = pltpu.CompilerParams(use_tc_tiling_on_sc=False)


def _deg_kernel(dst_hbm, ones_hbm, zeros_hbm, out_hbm, idx_v, ones_v, acc,
                isem, ssem):
    c = lax.axis_index("c")
    s = lax.axis_index("s")
    wid = c * NS + s

    pltpu.sync_copy(ones_hbm, ones_v.at[0])

    lo = s * RPS
    for q in range(8):
        pltpu.sync_copy(zeros_hbm, acc.at[pl.ds(lo + q * (RPS // 8),
                                                RPS // 8)])
    plsc.subcore_barrier()

    sb_lo = wid * DNSB // NW
    sb_hi = (wid + 1) * DNSB // NW
    count = sb_hi - sb_lo

    def stage_scatter(sb, p):
        pltpu.async_copy(dst_hbm.at[pl.ds(sb * DKCH, DKCH)], idx_v.at[p],
                         isem)
        pltpu.make_async_copy(dst_hbm.at[pl.ds(0, DKCH)], idx_v.at[p],
                              isem).wait()
        for j in range(DKCH):
            pltpu.async_copy(ones_v.at[0], acc.at[idx_v.at[p, j]],
                             ssem.at[p], add=True)

    def drain_scatter(p):
        for j in range(DKCH):
            pltpu.make_async_copy(ones_v.at[0], acc.at[idx_v.at[p, j]],
                                  ssem.at[p]).wait()

    def body(t, _):
        @pl.when(t >= 2)
        def _():
            drain_scatter((t - 2) % 2)

        @pl.when(t < count)
        def _():
            stage_scatter(sb_lo + t, t % 2)

        return ()

    lax.fori_loop(0, count + 2, body, (), unroll=False)

    plsc.subcore_barrier()
    pltpu.sync_copy(acc.at[pl.ds(lo, RPS)], out_hbm.at[c, pl.ds(lo, RPS)])


def _sc_degree(dst3, ones1, zeros1):
    return pl.kernel(
        _deg_kernel,
        out_type=jax.ShapeDtypeStruct((NC, NPAD, 1), jnp.float32),
        mesh=_mesh,
        scratch_types=[
            pltpu.VMEM((2, DKCH, CHUNK), jnp.int32),
            pltpu.VMEM((1, CHUNK, 1), jnp.float32),
            pltpu.VMEM_SHARED((NPAD, 1), jnp.float32),
            pltpu.SemaphoreType.DMA,
            pltpu.SemaphoreType.DMA((2,)),
        ],
        compiler_params=_sc_params,
    )(dst3, ones1, zeros1)


def _make_edge_kernel(d, kch):
    def body(src_hbm, dst_hbm, tab_hbm, zeros_hbm, out_hbm,
             isrc_v, idst_v, rows_v, acc, isem, gsem, ssem):
        c = lax.axis_index("c")
        s = lax.axis_index("s")
        wid = c * NS + s

        lo = s * RPS
        for q in range(8):
            pltpu.sync_copy(zeros_hbm, acc.at[pl.ds(lo + q * (RPS // 8),
                                                    RPS // 8)])
        plsc.subcore_barrier()

        nsb = N_EDGES // (CHUNK * kch)
        sb_lo = wid * nsb // NW
        sb_hi = (wid + 1) * nsb // NW
        count = sb_hi - sb_lo

        def stage_a(sb, p):
            pltpu.async_copy(src_hbm.at[pl.ds(sb * kch, kch)],
                             isrc_v.at[p], isem)
            pltpu.async_copy(dst_hbm.at[pl.ds(sb * kch, kch)],
                             idst_v.at[p], isem)
            pltpu.make_async_copy(src_hbm.at[pl.ds(0, kch)], isrc_v.at[p],
                                  isem).wait()
            pltpu.make_async_copy(dst_hbm.at[pl.ds(0, kch)], idst_v.at[p],
                                  isem).wait()
            for j in range(kch):
                pltpu.async_copy(tab_hbm.at[isrc_v.at[p, j]],
                                 rows_v.at[p, j], gsem.at[p])

        def stage_b(p):
            for j in range(kch):
                pltpu.make_async_copy(tab_hbm.at[isrc_v.at[p, j]],
                                      rows_v.at[p, j], gsem.at[p]).wait()
            for j in range(kch):
                pltpu.async_copy(rows_v.at[p, j], acc.at[idst_v.at[p, j]],
                                 ssem.at[p], add=True)

        def stage_c(p):
            for j in range(kch):
                pltpu.make_async_copy(rows_v.at[p, j],
                                      acc.at[idst_v.at[p, j]],
                                      ssem.at[p]).wait()

        def body_t(t, _):
            @pl.when(t >= 2)
            def _():
                stage_c((t - 2) % 2)

            @pl.when(t < count)
            def _():
                stage_a(sb_lo + t, t % 2)

            @pl.when((t >= 1) & (t <= count))
            def _():
                stage_b((t - 1) % 2)

            return ()

        lax.fori_loop(0, count + 2, body_t, (), unroll=False)

        plsc.subcore_barrier()
        pltpu.sync_copy(acc.at[pl.ds(lo, RPS)],
                        out_hbm.at[c, pl.ds(lo, RPS)])

    def call(src3, dst3, table, zeros2):
        return pl.kernel(
            body,
            out_type=jax.ShapeDtypeStruct((NC, NPAD, d), jnp.float32),
            mesh=_mesh,
            scratch_types=[
                pltpu.VMEM((2, kch, CHUNK), jnp.int32),
                pltpu.VMEM((2, kch, CHUNK), jnp.int32),
                pltpu.VMEM((2, kch, CHUNK, d), jnp.float32),
                pltpu.VMEM_SHARED((NPAD, d), jnp.float32),
                pltpu.SemaphoreType.DMA,
                pltpu.SemaphoreType.DMA((2,)),
                pltpu.SemaphoreType.DMA((2,)),
            ],
            compiler_params=_sc_params,
        )(src3, dst3, table, zeros2)

    return call


_edge_pass_16 = _make_edge_kernel(16, 5)


_edge_pass_8 = _make_edge_kernel(8, 10)


_RB = 5000


def _tc1_body(deg2, x, w1, dis_o, g1_o):
    deg = deg2[0] + deg2[1]
    dis = lax.rsqrt(deg)
    x_ = x[...]
    w = w1[...]
    h = x_[:, 0:1] * w[0:1, :]
    h = h + x_[:, 1:2] * w[1:2, :]
    h = h + x_[:, 2:3] * w[2:3, :]
    dis_o[...] = dis
    g1_o[...] = dis * h


def _tc1(deg2, x, w1):
    return pl.pallas_call(
        _tc1_body,
        grid=(N_NODES // _RB,),
        in_specs=[
            pl.BlockSpec((2, _RB, 1), lambda i: (0, i, 0)),
            pl.BlockSpec((_RB, 3), lambda i: (i, 0)),
            pl.BlockSpec((3, 16), lambda i: (0, 0)),
        ],
        out_specs=[
            pl.BlockSpec((_RB, 1), lambda i: (i, 0)),
            pl.BlockSpec((_RB, 16), lambda i: (i, 0)),
        ],
        out_shape=[
            jax.ShapeDtypeStruct((N_NODES, 1), jnp.float32),
            jax.ShapeDtypeStruct((N_NODES, 16), jnp.float32),
        ],
    )(deg2, x, w1)


def _tc2_body(acc2, g1, dis, b1, w2, g2_o):
    d = dis[...]
    out1 = jnp.maximum(d * (acc2[0] + acc2[1] + g1[...]) + b1[...], 0.0)
    h2 = jnp.dot(out1, w2[...], preferred_element_type=jnp.float32)
    g2_o[...] = d * h2


def _tc2(acc2, g1, dis, b1, w2p):
    return pl.pallas_call(
        _tc2_body,
        grid=(N_NODES // _RB,),
        in_specs=[
            pl.BlockSpec((2, _RB, 16), lambda i: (0, i, 0)),
            pl.BlockSpec((_RB, 16), lambda i: (i, 0)),
            pl.BlockSpec((_RB, 1), lambda i: (i, 0)),
            pl.BlockSpec((1, 16), lambda i: (0, 0)),
            pl.BlockSpec((16, 8), lambda i: (0, 0)),
        ],
        out_specs=pl.BlockSpec((_RB, 8), lambda i: (i, 0)),
        out_shape=jax.ShapeDtypeStruct((N_NODES, 8), jnp.float32),
    )(acc2, g1, dis, b1, w2p)


def _tc3_body(acc2, g2, dis, b2, out_o):
    out_o[...] = dis[...] * (acc2[0] + acc2[1] + g2[...]) + b2[...]


def _tc3(acc2, g2, dis, b2p):
    return pl.pallas_call(
        _tc3_body,
        grid=(N_NODES // _RB,),
        in_specs=[
            pl.BlockSpec((2, _RB, 8), lambda i: (0, i, 0)),
            pl.BlockSpec((_RB, 8), lambda i: (i, 0)),
            pl.BlockSpec((_RB, 1), lambda i: (i, 0)),
            pl.BlockSpec((1, 8), lambda i: (0, 0)),
        ],
        out_specs=pl.BlockSpec((_RB, 8), lambda i: (i, 0)),
        out_shape=jax.ShapeDtypeStruct((N_NODES, 8), jnp.float32),
    )(acc2, g2, dis, b2p)


def kernel(x, edge_index, W1, b1, W2, b2):
    src3 = edge_index[0].reshape(NCHUNKS, CHUNK)
    dst3 = edge_index[1].reshape(NCHUNKS, CHUNK)

    zeros1 = jnp.zeros((RPS // 8, 1), jnp.float32)
    zeros16 = jnp.zeros((RPS // 8, 16), jnp.float32)
    zeros8 = jnp.zeros((RPS // 8, 8), jnp.float32)
    ones1 = jnp.ones((CHUNK, 1), jnp.float32)

    deg2 = _sc_degree(dst3, ones1, zeros1)

    dis, g1 = _tc1(deg2, x, W1)

    acc1 = _edge_pass_16(src3, dst3, g1, zeros16)
    b1r = b1.reshape(1, 16)
    w2p = jnp.pad(W2, ((0, 0), (0, 1)))
    g2 = _tc2(acc1, g1, dis, b1r, w2p)

    acc2 = _edge_pass_8(src3, dst3, g2, zeros8)
    b2p = jnp.pad(b2, (0, 1)).reshape(1, 8)
    out = _tc3(acc2, g2, dis, b2p)
    return out[:, :7]


# speedup vs baseline: 117.6221x; 1.0090x over previous
import jax
import jax.numpy as jnp
from jax import lax
from jax.experimental import pallas as pl
from jax.experimental.pallas import tpu as pltpu
from jax.experimental.pallas import tpu_sc as plsc

N_NODES = 100000
N_EDGES = 6400000

NC = 2
NS = 16
NW = NC * NS

CHUNK = 128
KCH = 5
SB = CHUNK * KCH
NSB = N_EDGES // SB

NCHUNKS = N_EDGES // CHUNK

DKCH = 40
DNSB = N_EDGES // (CHUNK * DKCH)

NPAD = 100096
RPS = NPAD // NS

_mesh = plsc.VectorSubcoreMesh(core_axis_name="c", subcore_axis_name="s")
_sc_params = pltpu.CompilerParams(use_tc_tiling_on_sc=False)


def _deg_kernel(dst_hbm, ones_hbm, zeros_hbm, out_hbm, idx_v, ones_v, acc,
                isem, ssem):
    c = lax.axis_index("c")
    s = lax.axis_index("s")
    wid = c * NS + s

    pltpu.sync_copy(ones_hbm, ones_v.at[0])

    lo = s * RPS
    for q in range(8):
        pltpu.sync_copy(zeros_hbm, acc.at[pl.ds(lo + q * (RPS // 8),
                                                RPS // 8)])
    plsc.subcore_barrier()

    sb_lo = wid * DNSB // NW
    sb_hi = (wid + 1) * DNSB // NW
    count = sb_hi - sb_lo

    def stage_scatter(sb, p):
        pltpu.async_copy(dst_hbm.at[pl.ds(sb * DKCH, DKCH)], idx_v.at[p],
                         isem)
        pltpu.make_async_copy(dst_hbm.at[pl.ds(0, DKCH)], idx_v.at[p],
                              isem).wait()
        for j in range(DKCH):
            pltpu.async_copy(ones_v.at[0], acc.at[idx_v.at[p, j]],
                             ssem.at[p], add=True)

    def drain_scatter(p):
        for j in range(DKCH):
            pltpu.make_async_copy(ones_v.at[0], acc.at[idx_v.at[p, j]],
                                  ssem.at[p]).wait()

    def body(t, _):
        @pl.when(t >= 2)
        def _():
            drain_scatter((t - 2) % 2)

        @pl.when(t < count)
        def _():
            stage_scatter(sb_lo + t, t % 2)

        return ()

    lax.fori_loop(0, count + 2, body, (), unroll=False)

    plsc.subcore_barrier()
    pltpu.sync_copy(acc.at[pl.ds(lo, RPS)], out_hbm.at[c, pl.ds(lo, RPS)])


def _sc_degree(dst3, ones1, zeros1):
    return pl.kernel(
        _deg_kernel,
        out_type=jax.ShapeDtypeStruct((NC, NPAD, 1), jnp.float32),
        mesh=_mesh,
        scratch_types=[
            pltpu.VMEM((2, DKCH, CHUNK), jnp.int32),
            pltpu.VMEM((1, CHUNK, 1), jnp.float32),
            pltpu.VMEM_SHARED((NPAD, 1), jnp.float32),
            pltpu.SemaphoreType.DMA,
            pltpu.SemaphoreType.DMA((2,)),
        ],
        compiler_params=_sc_params,
    )(dst3, ones1, zeros1)


def _make_edge_kernel(d, kch):
    def body(src_hbm, dst_hbm, tab_hbm, zeros_hbm, out_hbm,
             isrc_v, idst_v, rows_v, acc, isem, gsem, ssem):
        c = lax.axis_index("c")
        s = lax.axis_index("s")
        wid = c * NS + s

        lo = s * RPS
        for q in range(8):
            pltpu.sync_copy(zeros_hbm, acc.at[pl.ds(lo + q * (RPS // 8),
                                                    RPS // 8)])
        plsc.subcore_barrier()

        nsb = N_EDGES // (CHUNK * kch)
        sb_lo = wid * nsb // NW
        sb_hi = (wid + 1) * nsb // NW
        count = sb_hi - sb_lo

        def stage_a(sb, p):
            pltpu.async_copy(src_hbm.at[pl.ds(sb * kch, kch)],
                             isrc_v.at[p], isem)
            pltpu.async_copy(dst_hbm.at[pl.ds(sb * kch, kch)],
                             idst_v.at[p], isem)
            pltpu.make_async_copy(src_hbm.at[pl.ds(0, kch)], isrc_v.at[p],
                                  isem).wait()
            pltpu.make_async_copy(dst_hbm.at[pl.ds(0, kch)], idst_v.at[p],
                                  isem).wait()
            for j in range(kch):
                pltpu.async_copy(tab_hbm.at[isrc_v.at[p, j]],
                                 rows_v.at[p, j], gsem.at[p])

        def stage_b(p):
            for j in range(kch):
                pltpu.make_async_copy(tab_hbm.at[isrc_v.at[p, j]],
                                      rows_v.at[p, j], gsem.at[p]).wait()
            for j in range(kch):
                pltpu.async_copy(rows_v.at[p, j], acc.at[idst_v.at[p, j]],
                                 ssem.at[p], add=True)

        def stage_c(p):
            for j in range(kch):
                pltpu.make_async_copy(rows_v.at[p, j],
                                      acc.at[idst_v.at[p, j]],
                                      ssem.at[p]).wait()

        def body_t(t, _):
            @pl.when(t >= 2)
            def _():
                stage_c((t - 2) % 2)

            @pl.when(t < count)
            def _():
                stage_a(sb_lo + t, t % 2)

            @pl.when((t >= 1) & (t <= count))
            def _():
                stage_b((t - 1) % 2)

            return ()

        lax.fori_loop(0, count + 2, body_t, (), unroll=False)

        plsc.subcore_barrier()
        pltpu.sync_copy(acc.at[pl.ds(lo, RPS)],
                        out_hbm.at[c, pl.ds(lo, RPS)])

    def call(src3, dst3, table, zeros2):
        return pl.kernel(
            body,
            out_type=jax.ShapeDtypeStruct((NC, NPAD, d), jnp.float32),
            mesh=_mesh,
            scratch_types=[
                pltpu.VMEM((2, kch, CHUNK), jnp.int32),
                pltpu.VMEM((2, kch, CHUNK), jnp.int32),
                pltpu.VMEM((2, kch, CHUNK, d), jnp.float32),
                pltpu.VMEM_SHARED((NPAD, d), jnp.float32),
                pltpu.SemaphoreType.DMA,
                pltpu.SemaphoreType.DMA((2,)),
                pltpu.SemaphoreType.DMA((2,)),
            ],
            compiler_params=_sc_params,
        )(src3, dst3, table, zeros2)

    return call


_edge_pass_16 = _make_edge_kernel(16, 5)


_edge_pass_8 = _make_edge_kernel(8, 20)


_RB = 5000


def _tc1_body(deg2, x, w1, dis_o, g1_o):
    deg = deg2[0] + deg2[1]
    dis = lax.rsqrt(deg)
    x_ = x[...]
    w = w1[...]
    h = x_[:, 0:1] * w[0:1, :]
    h = h + x_[:, 1:2] * w[1:2, :]
    h = h + x_[:, 2:3] * w[2:3, :]
    dis_o[...] = dis
    g1_o[...] = dis * h


def _tc1(deg2, x, w1):
    return pl.pallas_call(
        _tc1_body,
        grid=(N_NODES // _RB,),
        in_specs=[
            pl.BlockSpec((2, _RB, 1), lambda i: (0, i, 0)),
            pl.BlockSpec((_RB, 3), lambda i: (i, 0)),
            pl.BlockSpec((3, 16), lambda i: (0, 0)),
        ],
        out_specs=[
            pl.BlockSpec((_RB, 1), lambda i: (i, 0)),
            pl.BlockSpec((_RB, 16), lambda i: (i, 0)),
        ],
        out_shape=[
            jax.ShapeDtypeStruct((N_NODES, 1), jnp.float32),
            jax.ShapeDtypeStruct((N_NODES, 16), jnp.float32),
        ],
    )(deg2, x, w1)


def _tc2_body(acc2, g1, dis, b1, w2, g2_o):
    d = dis[...]
    out1 = jnp.maximum(d * (acc2[0] + acc2[1] + g1[...]) + b1[...], 0.0)
    h2 = jnp.dot(out1, w2[...], preferred_element_type=jnp.float32)
    g2_o[...] = d * h2


def _tc2(acc2, g1, dis, b1, w2p):
    return pl.pallas_call(
        _tc2_body,
        grid=(N_NODES // _RB,),
        in_specs=[
            pl.BlockSpec((2, _RB, 16), lambda i: (0, i, 0)),
            pl.BlockSpec((_RB, 16), lambda i: (i, 0)),
            pl.BlockSpec((_RB, 1), lambda i: (i, 0)),
            pl.BlockSpec((1, 16), lambda i: (0, 0)),
            pl.BlockSpec((16, 8), lambda i: (0, 0)),
        ],
        out_specs=pl.BlockSpec((_RB, 8), lambda i: (i, 0)),
        out_shape=jax.ShapeDtypeStruct((N_NODES, 8), jnp.float32),
    )(acc2, g1, dis, b1, w2p)


def _tc3_body(acc2, g2, dis, b2, out_o):
    out_o[...] = dis[...] * (acc2[0] + acc2[1] + g2[...]) + b2[...]


def _tc3(acc2, g2, dis, b2p):
    return pl.pallas_call(
        _tc3_body,
        grid=(N_NODES // _RB,),
        in_specs=[
            pl.BlockSpec((2, _RB, 8), lambda i: (0, i, 0)),
            pl.BlockSpec((_RB, 8), lambda i: (i, 0)),
            pl.BlockSpec((_RB, 1), lambda i: (i, 0)),
            pl.BlockSpec((1, 8), lambda i: (0, 0)),
        ],
        out_specs=pl.BlockSpec((_RB, 8), lambda i: (i, 0)),
        out_shape=jax.ShapeDtypeStruct((N_NODES, 8), jnp.float32),
    )(acc2, g2, dis, b2p)


def kernel(x, edge_index, W1, b1, W2, b2):
    src3 = edge_index[0].reshape(NCHUNKS, CHUNK)
    dst3 = edge_index[1].reshape(NCHUNKS, CHUNK)

    zeros1 = jnp.zeros((RPS // 8, 1), jnp.float32)
    zeros16 = jnp.zeros((RPS // 8, 16), jnp.float32)
    zeros8 = jnp.zeros((RPS // 8, 8), jnp.float32)
    ones1 = jnp.ones((CHUNK, 1), jnp.float32)

    deg2 = _sc_degree(dst3, ones1, zeros1)

    dis, g1 = _tc1(deg2, x, W1)

    acc1 = _edge_pass_16(src3, dst3, g1, zeros16)
    b1r = b1.reshape(1, 16)
    w2p = jnp.pad(W2, ((0, 0), (0, 1)))
    g2 = _tc2(acc1, g1, dis, b1r, w2p)

    acc2 = _edge_pass_8(src3, dst3, g2, zeros8)
    b2p = jnp.pad(b2, (0, 1)).reshape(1, 8)
    out = _tc3(acc2, g2, dis, b2p)
    return out[:, :7]
